# Initial kernel scaffold; baseline (speedup 1.0000x reference)
import jax
import jax.numpy as jnp
from jax.experimental import pallas as pl


def kernel(x, edge_index, W1, b1, W2, b2):
    raise NotImplementedError("write your pallas kernel here")


# speedup vs baseline: 35.3655x; 35.3655x over previous
import functools

import jax
import jax.numpy as jnp
from jax import lax
from jax.experimental import pallas as pl
from jax.experimental.pallas import tpu as pltpu
from jax.experimental.pallas import tpu_sc as plsc

N = 10000
NPAD = 10240
HID = 8
NC = 2
NS = 16
NW = NC * NS
CH = 128
ROWS_PT = NPAD // NS
BR = 1024


def _sc_mesh():
    return plsc.VectorSubcoreMesh(core_axis_name="c", subcore_axis_name="s")


_SC_PARAMS = pltpu.CompilerParams(use_tc_tiling_on_sc=False)


@functools.lru_cache(maxsize=None)
def _deg_call(k_chunks):
    @functools.partial(
        pl.kernel,
        mesh=_sc_mesh(),
        out_type=jax.ShapeDtypeStruct((NC, NPAD, HID), jnp.float32),
        compiler_params=_SC_PARAMS,
        scratch_types=[
            pltpu.VMEM((k_chunks, CH), jnp.int32),
            pltpu.VMEM((CH, HID), jnp.float32),
            pltpu.VMEM_SHARED((NPAD, HID), jnp.float32),
        ],
    )
    def deg_kernel(dst_hbm, zeros_hbm, ones_hbm, out_hbm, idx_v, ones_v, accum):
        c = lax.axis_index("c")
        s = lax.axis_index("s")
        w = s * NC + c
        r0 = s * ROWS_PT
        pltpu.sync_copy(zeros_hbm.at[pl.ds(r0, ROWS_PT)],
                        accum.at[pl.ds(r0, ROWS_PT)])
        pltpu.sync_copy(ones_hbm, ones_v)
        pltpu.sync_copy(dst_hbm.at[w], idx_v)
        plsc.subcore_barrier()

        def body(j, carry):
            pltpu.sync_copy(ones_v, accum.at[idx_v.at[j]], add=True)
            return carry

        lax.fori_loop(0, k_chunks, body, 0)
        plsc.subcore_barrier()
        pltpu.sync_copy(accum.at[pl.ds(r0, ROWS_PT)],
                        out_hbm.at[c, pl.ds(r0, ROWS_PT)])

    return deg_kernel


@functools.lru_cache(maxsize=None)
def _agg_call(k_chunks):
    @functools.partial(
        pl.kernel,
        mesh=_sc_mesh(),
        out_type=jax.ShapeDtypeStruct((NC, NPAD, HID), jnp.float32),
        compiler_params=_SC_PARAMS,
        scratch_types=[
            pltpu.VMEM((k_chunks, CH), jnp.int32),
            pltpu.VMEM((k_chunks, CH), jnp.int32),
            pltpu.VMEM((CH, HID), jnp.float32),
            pltpu.VMEM_SHARED((NPAD, HID), jnp.float32),
        ],
    )
    def agg_kernel(tab_hbm, src_hbm, dst_hbm, zeros_hbm, out_hbm,
                   sidx_v, didx_v, rows_v, accum):
        c = lax.axis_index("c")
        s = lax.axis_index("s")
        w = s * NC + c
        r0 = s * ROWS_PT
        pltpu.sync_copy(zeros_hbm.at[pl.ds(r0, ROWS_PT)],
                        accum.at[pl.ds(r0, ROWS_PT)])
        pltpu.sync_copy(src_hbm.at[w], sidx_v)
        pltpu.sync_copy(dst_hbm.at[w], didx_v)
        plsc.subcore_barrier()

        def body(j, carry):
            pltpu.sync_copy(tab_hbm.at[sidx_v.at[j]], rows_v)
            pltpu.sync_copy(rows_v, accum.at[didx_v.at[j]], add=True)
            return carry

        lax.fori_loop(0, k_chunks, body, 0)
        plsc.subcore_barrier()
        pltpu.sync_copy(accum.at[pl.ds(r0, ROWS_PT)],
                        out_hbm.at[c, pl.ds(r0, ROWS_PT)])

    return agg_kernel


def _tc1_body(x_ref, w1_ref, degp_ref, y1_ref, dinv_ref):
    degs = degp_ref[0] + degp_ref[1]
    dinv = lax.rsqrt(degs + 1.0)
    xw = jnp.dot(x_ref[...], w1_ref[...], preferred_element_type=jnp.float32)
    y1_ref[...] = xw * dinv
    dinv_ref[...] = dinv


def _tc2_body(aggp_ref, y1_ref, dinv_ref, b1_ref, y2_ref):
    dinv = dinv_ref[...]
    agg = (aggp_ref[0] + aggp_ref[1] + y1_ref[...]) * dinv + b1_ref[...]
    h = jnp.where(agg > 0, agg, jnp.exp(jnp.minimum(agg, 0.0)) - 1.0)
    y2_ref[...] = h * dinv


def _tc3_body(aggp_ref, y2_ref, dinv_ref, w2_ref, b2_ref, out_ref):
    t = (aggp_ref[0] + aggp_ref[1] + y2_ref[...]) * dinv_ref[...]
    z = jnp.dot(t, w2_ref[...], preferred_element_type=jnp.float32)
    z = z + b2_ref[...]
    m = jnp.max(z, axis=1, keepdims=True)
    lse = m + jnp.log(jnp.sum(jnp.exp(z - m), axis=1, keepdims=True))
    out_ref[...] = z - lse


def _row_block(width):
    return pl.BlockSpec((BR, width), lambda r: (r, 0))


def _pair_block(width):
    return pl.BlockSpec((NC, BR, width), lambda r: (0, r, 0))


def _full_block(a, b):
    return pl.BlockSpec((a, b), lambda r: (0, 0))


def _tc1(x_pad, w1, degp):
    return pl.pallas_call(
        _tc1_body,
        grid=(NPAD // BR,),
        in_specs=[_row_block(128), _full_block(128, HID), _pair_block(HID)],
        out_specs=[_row_block(HID), _row_block(HID)],
        out_shape=[jax.ShapeDtypeStruct((NPAD, HID), jnp.float32)] * 2,
    )(x_pad, w1, degp)


def _tc2(aggp, y1, dinv8, b1):
    return pl.pallas_call(
        _tc2_body,
        grid=(NPAD // BR,),
        in_specs=[_pair_block(HID), _row_block(HID), _row_block(HID),
                  _full_block(1, HID)],
        out_specs=_row_block(HID),
        out_shape=jax.ShapeDtypeStruct((NPAD, HID), jnp.float32),
    )(aggp, y1, dinv8, b1)


def _tc3(aggp, y2, dinv8, w2, b2, ncls):
    return pl.pallas_call(
        _tc3_body,
        grid=(NPAD // BR,),
        in_specs=[_pair_block(HID), _row_block(HID), _row_block(HID),
                  _full_block(HID, ncls), _full_block(1, ncls)],
        out_specs=_row_block(ncls),
        out_shape=jax.ShapeDtypeStruct((NPAD, ncls), jnp.float32),
    )(aggp, y2, dinv8, w2, b2)


def kernel(x, edge_index, W1, b1, W2, b2):
    n_edges = edge_index.shape[1]
    k_chunks = -(-n_edges // (NW * CH))
    e_pad = NW * CH * k_chunks
    ncls = W2.shape[1]

    pad_e = jnp.full((2, e_pad - n_edges), N, dtype=jnp.int32)
    ei = jnp.concatenate([edge_index.astype(jnp.int32), pad_e], axis=1)
    src3 = ei[0].reshape(NW, k_chunks, CH)
    dst3 = ei[1].reshape(NW, k_chunks, CH)

    x_pad = jnp.pad(x, ((0, NPAD - N), (0, 0)))
    zeros = jnp.zeros((NPAD, HID), jnp.float32)
    ones = jnp.ones((CH, HID), jnp.float32)

    degp = _deg_call(k_chunks)(dst3, zeros, ones)
    y1, dinv8 = _tc1(x_pad, W1, degp)
    aggp1 = _agg_call(k_chunks)(y1, src3, dst3, zeros)
    y2 = _tc2(aggp1, y1, dinv8, b1.reshape(1, HID))
    aggp2 = _agg_call(k_chunks)(y2, src3, dst3, zeros)
    out = _tc3(aggp2, y2, dinv8, W2, b2.reshape(1, ncls), ncls)
    return out[:N]
